# baseline (speedup 1.0000x reference)
import functools
import numpy as np
import jax
import jax.numpy as jnp
from jax import lax
from jax.experimental import pallas as pl
from jax.experimental.pallas import tpu as pltpu
from jax.experimental.pallas import tpu_sc as plsc

_B = 64
_C = 80
_D = 256
_P = _B * (_B - 1) // 2
_G = _C // 2
_CP = 128
_PP = 2048
_NW = 32
_RPW = _PP // _NW

_i0, _i1 = np.triu_indices(_B, k=1)
_flat = (_i0 * _B + _i1).astype(np.int32)
_IDX = jnp.asarray(np.concatenate([_flat, np.zeros(_PP - _P, np.int32)]))


def _tc_body(x_ref, out_ref):
    x = x_ref[...]
    xt = jnp.transpose(x, (1, 0, 2))
    a = jnp.concatenate([xt[:_G], xt[_G:]], axis=1)
    gram2 = jax.lax.dot_general(
        a, a,
        dimension_numbers=(((2,), (2,)), ((0,), (0,))),
        preferred_element_type=jnp.float32,
    )
    row = jax.lax.broadcasted_iota(jnp.int32, (_G, 2 * _B, 2 * _B), 1)
    col = jax.lax.broadcasted_iota(jnp.int32, (_G, 2 * _B, 2 * _B), 2)
    diag = jnp.sum(jnp.where(row == col, gram2, 0.0), axis=2)
    r = 1.0 / jnp.maximum(jnp.sqrt(diag), 3.1622776601683795e-05)
    dist2 = gram2 * (r[:, :, None] * r[:, None, :])
    dist = jnp.concatenate([dist2[:, :_B, :_B], dist2[:, _B:, _B:]], axis=0)
    dist_t = jnp.transpose(dist, (1, 2, 0))
    flat = dist_t.reshape(_B * _B, _C)
    out_ref[...] = jnp.concatenate(
        [flat, jnp.zeros((_B * _B, _CP - _C), jnp.float32)], axis=1)


@functools.lru_cache(maxsize=1)
def _make_sc_gather():
    mesh = plsc.VectorSubcoreMesh(core_axis_name="c", subcore_axis_name="s")

    @functools.partial(
        pl.kernel,
        mesh=mesh,
        out_type=jax.ShapeDtypeStruct((_PP, _CP), jnp.float32),
        scratch_types=[
            pltpu.VMEM((_RPW,), jnp.int32),
            pltpu.VMEM((_RPW, _CP), jnp.float32),
            pltpu.SemaphoreType.DMA,
        ],
    )
    def _sc_gather(dist_hbm, idx_hbm, out_hbm, idx_v, rows_v, sem):
        wid = lax.axis_index("s") * 2 + lax.axis_index("c")
        base = wid * _RPW
        pltpu.sync_copy(idx_hbm.at[pl.ds(base, _RPW)], idx_v)
        pltpu.async_copy(dist_hbm.at[idx_v], rows_v, sem).wait()
        pltpu.sync_copy(rows_v, out_hbm.at[pl.ds(base, _RPW)])

    return _sc_gather


def kernel(input, target, meter):
    del target, meter
    dist_flat = pl.pallas_call(
        _tc_body,
        out_shape=jax.ShapeDtypeStruct((_B * _B, _CP), jnp.float32),
    )(input)
    padded = _make_sc_gather()(dist_flat, _IDX)
    return padded[:_P, :_C]


# speedup vs baseline: 2.7686x; 2.7686x over previous
import jax
import jax.numpy as jnp
from jax.experimental import pallas as pl
from jax.experimental.pallas import tpu as pltpu

_B = 64
_C = 80
_D = 256
_P = _B * (_B - 1) // 2
_G = _C // 2
_DB = 128


def _partial_gram(buf):
    xt = jnp.transpose(buf, (1, 0, 2))
    a = jnp.concatenate([xt[:_G], xt[_G:]], axis=1)
    return jax.lax.dot_general(
        a, a,
        dimension_numbers=(((2,), (2,)), ((0,), (0,))),
        preferred_element_type=jnp.float32,
    )


def _cosine_body(x_hbm, out_ref, buf0, buf1, sem0, sem1):
    cp0 = pltpu.make_async_copy(x_hbm.at[:, :, pl.ds(0, _DB)], buf0, sem0)
    cp1 = pltpu.make_async_copy(x_hbm.at[:, :, pl.ds(_DB, _DB)], buf1, sem1)
    cp0.start()
    cp1.start()
    cp0.wait()
    part0 = _partial_gram(buf0[...])
    cp1.wait()
    gram2 = part0 + _partial_gram(buf1[...])

    row = jax.lax.broadcasted_iota(jnp.int32, (_G, 2 * _B, 2 * _B), 1)
    col = jax.lax.broadcasted_iota(jnp.int32, (_G, 2 * _B, 2 * _B), 2)
    diag = jnp.sum(jnp.where(row == col, gram2, 0.0), axis=2)
    r = 1.0 / jnp.maximum(jnp.sqrt(diag), 3.1622776601683795e-05)
    dist2 = gram2 * (r[:, :, None] * r[:, None, :])
    dist = jnp.concatenate([dist2[:, :_B, :_B], dist2[:, _B:, _B:]], axis=0)
    dist_t = jnp.transpose(dist, (1, 2, 0))

    for i in range(_B - 2, -1, -1):
        start = 62 * i - (i * (i - 1)) // 2
        blk = jax.lax.slice(dist_t, (i, 1, 0), (i + 1, _B, _C))
        out_ref[start:start + _B - 1, :] = blk.reshape(_B - 1, _C)


def kernel(input, target, meter):
    del target, meter
    return pl.pallas_call(
        _cosine_body,
        in_specs=[pl.BlockSpec(memory_space=pltpu.MemorySpace.HBM)],
        out_shape=jax.ShapeDtypeStruct((_P, _C), jnp.float32),
        scratch_shapes=[
            pltpu.VMEM((_B, _C, _DB), jnp.float32),
            pltpu.VMEM((_B, _C, _DB), jnp.float32),
            pltpu.SemaphoreType.DMA,
            pltpu.SemaphoreType.DMA,
        ],
    )(input)


# speedup vs baseline: 2.9608x; 1.0694x over previous
import jax
import jax.numpy as jnp
from jax.experimental import pallas as pl
from jax.experimental.pallas import tpu as pltpu

_B = 64
_C = 80
_D = 256
_P = _B * (_B - 1) // 2
_G = _C // 2


def _cosine_body(x_ref, out_ref):
    x = x_ref[...]
    xt = jnp.transpose(x, (1, 0, 2))
    a = jnp.concatenate([xt[:_G], xt[_G:]], axis=1)
    gram2 = jax.lax.dot_general(
        a, a,
        dimension_numbers=(((2,), (2,)), ((0,), (0,))),
        preferred_element_type=jnp.float32,
    )
    row = jax.lax.broadcasted_iota(jnp.int32, (_G, 2 * _B, 2 * _B), 1)
    col = jax.lax.broadcasted_iota(jnp.int32, (_G, 2 * _B, 2 * _B), 2)
    diag = jnp.sum(jnp.where(row == col, gram2, 0.0), axis=2)
    norm = jnp.sqrt(diag)
    denom = jnp.maximum(norm[:, :, None] * norm[:, None, :], 1e-9)
    dist2 = gram2 / denom
    dist = jnp.concatenate(
        [dist2[:, :_B, :_B], dist2[:, _B:, _B:]], axis=0)
    dist_t = jnp.transpose(dist, (1, 2, 0))

    for i in range(_B - 2, -1, -1):
        start = 62 * i - (i * (i - 1)) // 2
        blk = jax.lax.slice(dist_t, (i, 1, 0), (i + 1, _B, _C))
        out_ref[start:start + _B - 1, :] = blk.reshape(_B - 1, _C)


def kernel(input, target, meter):
    del target, meter
    return pl.pallas_call(
        _cosine_body,
        out_shape=jax.ShapeDtypeStruct((_P, _C), jnp.float32),
    )(input)
